# Initial kernel scaffold; baseline (speedup 1.0000x reference)
import jax
import jax.numpy as jnp
from jax.experimental import pallas as pl


def kernel(feature, xyz, neigh_idx, p):
    raise NotImplementedError("write your pallas kernel here")


# speedup vs baseline: 7.7372x; 7.7372x over previous
import functools

import jax
import jax.numpy as jnp
from jax import lax
from jax.experimental import pallas as pl
from jax.experimental.pallas import tpu as pltpu
from jax.experimental.pallas import tpu_sc as plsc

_BN = 400

_pcall = pl.pallas_call


def _leaky(x, slope):
    return jnp.where(x >= 0, x, slope * x)


def _sc_gather(table, idx):
    M = idx.shape[0]
    D = table.shape[1]
    info = plsc.get_sparse_core_info()
    nw = info.num_cores * info.num_subcores
    cs = 5000
    n_chunks = M // cs
    mesh = plsc.VectorSubcoreMesh(core_axis_name="c", subcore_axis_name="s")

    @functools.partial(
        pl.kernel,
        mesh=mesh,
        compiler_params=pltpu.CompilerParams(use_tc_tiling_on_sc=False),
        out_type=jax.ShapeDtypeStruct((M, D), jnp.float32),
        scratch_types=[
            pltpu.VMEM((cs,), jnp.int32),
            pltpu.VMEM((cs, D), jnp.float32),
            pltpu.SemaphoreType.DMA,
        ],
    )
    def k(table_hbm, idx_hbm, out_hbm, idx_v, rows_v, sem):
        wid = lax.axis_index("s") * info.num_cores + lax.axis_index("c")

        def body(j, carry):
            cid = j * nw + wid

            @pl.when(cid < n_chunks)
            def _():
                off = cid * cs
                pltpu.sync_copy(idx_hbm.at[pl.ds(off, cs)], idx_v)
                pltpu.async_copy(table_hbm.at[idx_v], rows_v, sem).wait()
                pltpu.sync_copy(rows_v, out_hbm.at[pl.ds(off, cs)])

            return carry

        lax.fori_loop(0, pl.cdiv(n_chunks, nw), body, 0)

    return k(table, idx)


def _acc(ref, upd):
    @pl.when(pl.program_id(0) == 0)
    def _():
        ref[...] = upd

    @pl.when(pl.program_id(0) > 0)
    def _():
        ref[...] += upd


def _stat_rows(y, c):
    return jnp.concatenate(
        [jnp.sum(y, 0)[None], jnp.sum(y * y, 0)[None],
         jnp.zeros((6, c), jnp.float32)], 0)


def _bn_affine(st, g, be, count):
    m = st[0] / count
    v = st[1] / count - m * m
    a = g / jnp.sqrt(v + 1e-6)
    return (a[None, :], (be - a * m)[None, :])


def _k1_body(feat_ref, w1t_ref, wst_ref, y1_ref, st1_ref, sts_ref):
    x = feat_ref[...]
    y1 = jnp.dot(x, w1t_ref[...], preferred_element_type=jnp.float32)
    ys = jnp.dot(x, wst_ref[...], preferred_element_type=jnp.float32)
    y1_ref[...] = y1
    _acc(st1_ref, _stat_rows(y1, 8))
    _acc(sts_ref, _stat_rows(ys, 32))


def _k2_body(y1_ref, xyz_ref, a_ref, c_ref, t1_ref):
    fpc = _leaky(a_ref[...] * y1_ref[...] + c_ref[...], 0.2)
    t1_ref[...] = jnp.concatenate(
        [xyz_ref[...], fpc, jnp.zeros((_BN, 5), jnp.float32)], axis=1)


def _fxyz_k(gk, rep):
    nx = gk[:, 0:3]
    rel = rep - nx
    dist = jnp.sqrt(jnp.sum(rel * rel, axis=1, keepdims=True))
    return jnp.concatenate(
        [dist, rel, rep, nx, jnp.zeros((_BN, 6), jnp.float32)], axis=1)


def _k3_body(g1_ref, t1_ref, wb1t_ref, st_ref, *, K):
    t1 = t1_ref[...]
    rep = t1[:, 0:3]
    w = wb1t_ref[...]
    s = jnp.zeros((8,), jnp.float32)
    ss = jnp.zeros((8,), jnp.float32)
    for k in range(K):
        fx = _fxyz_k(g1_ref[k], rep)
        yb1 = jnp.dot(fx, w, preferred_element_type=jnp.float32)
        s = s + jnp.sum(yb1, 0)
        ss = ss + jnp.sum(yb1 * yb1, 0)
    _acc(st_ref, jnp.concatenate(
        [s[None], ss[None], jnp.zeros((6, 8), jnp.float32)], 0))


def _softmax_pool(fc_list, att_list):
    m = att_list[0]
    for a in att_list[1:]:
        m = jnp.maximum(m, a)
    e_list = [jnp.exp(a - m) for a in att_list]
    z = e_list[0]
    for e in e_list[1:]:
        z = z + e
    s = fc_list[0] * e_list[0]
    for fc, e in zip(fc_list[1:], e_list[1:]):
        s = s + fc * e
    return s / z


def _k4_body(g1_ref, t1_ref, wb1t_ref, ab_ref, cb_ref, fct_ref, apwt_ref,
             wdt_ref, yc_ref, yd_ref, stc_ref, std_ref, *, K):
    t1 = t1_ref[...]
    rep = t1[:, 0:3]
    w = wb1t_ref[...]
    ab = ab_ref[...]
    cb = cb_ref[...]
    fct = fct_ref[...]
    wdt = wdt_ref[...]
    fc_list = []
    att_list = []
    sd = jnp.zeros((8,), jnp.float32)
    ssd = jnp.zeros((8,), jnp.float32)
    for k in range(K):
        gk = g1_ref[k]
        fx = _fxyz_k(gk, rep)
        yb1 = jnp.dot(fx, w, preferred_element_type=jnp.float32)
        fxyz1 = _leaky(ab * yb1 + cb, 0.2)
        fn = gk[:, 3:11]
        fck = jnp.concatenate([fn, fxyz1], axis=1)
        fc_list.append(fck)
        att_list.append(jnp.dot(fck, fct, preferred_element_type=jnp.float32))
        ydk = jnp.dot(fxyz1, wdt, preferred_element_type=jnp.float32)
        yd_ref[k] = ydk
        sd = sd + jnp.sum(ydk, 0)
        ssd = ssd + jnp.sum(ydk * ydk, 0)
    s1 = _softmax_pool(fc_list, att_list)
    yc = jnp.dot(s1, apwt_ref[...], preferred_element_type=jnp.float32)
    yc_ref[...] = yc
    _acc(stc_ref, _stat_rows(yc, 8))
    _acc(std_ref, jnp.concatenate(
        [sd[None], ssd[None], jnp.zeros((6, 8), jnp.float32)], 0))


def _k5_body(yc_ref, a_ref, c_ref, t2_ref):
    t2_ref[...] = _leaky(a_ref[...] * yc_ref[...] + c_ref[...], 0.2)


def _k6_body(g2_ref, yd_ref, ad_ref, cd_ref, fct_ref, apwt_ref,
             ye_ref, ste_ref, *, K):
    ad = ad_ref[...]
    cd = cd_ref[...]
    fct = fct_ref[...]
    fc_list = []
    att_list = []
    for k in range(K):
        fxyz2 = _leaky(ad * yd_ref[k] + cd, 0.2)
        fck = jnp.concatenate([g2_ref[k], fxyz2], axis=1)
        fc_list.append(fck)
        att_list.append(jnp.dot(fck, fct, preferred_element_type=jnp.float32))
    s2 = _softmax_pool(fc_list, att_list)
    ye = jnp.dot(s2, apwt_ref[...], preferred_element_type=jnp.float32)
    ye_ref[...] = ye
    _acc(ste_ref, _stat_rows(ye, 16))


def _k7_body(ye_ref, ae_ref, ce_ref, w2t_ref, yf_ref, stf_ref):
    fpc2 = _leaky(ae_ref[...] * ye_ref[...] + ce_ref[...], 0.2)
    yf = jnp.dot(fpc2, w2t_ref[...], preferred_element_type=jnp.float32)
    yf_ref[...] = yf
    _acc(stf_ref, _stat_rows(yf, 32))


def _k8_body(yf_ref, feat_ref, wst_ref, af_ref, cf_ref, as_ref, cs_ref,
             out_ref):
    ys = jnp.dot(feat_ref[...], wst_ref[...], preferred_element_type=jnp.float32)
    pre = (af_ref[...] * yf_ref[...] + cf_ref[...]
           + as_ref[...] * ys + cs_ref[...])
    out_ref[...] = _leaky(pre, 0.01)


def kernel(feature, xyz, neigh_idx, p):
    N = feature.shape[2]
    K = neigh_idx.shape[2]
    grid = (N // _BN,)

    feat = jnp.transpose(feature[0, :, :, 0])
    xyz0 = xyz[0]
    idxt = jnp.transpose(neigh_idx[0]).astype(jnp.int32).reshape(-1)

    w1t = jnp.transpose(p['w1'])
    wst = jnp.transpose(p['ws'])
    wb1t = jnp.concatenate(
        [jnp.transpose(p['bb_w1']), jnp.zeros((6, 8), jnp.float32)], 0)
    fct1 = jnp.transpose(p['ap1_fc'])
    apw1t = jnp.transpose(p['ap1_w'])
    wdt = jnp.transpose(p['bb_w2'])
    fct2 = jnp.transpose(p['ap2_fc'])
    apw2t = jnp.transpose(p['ap2_w'])
    w2t = jnp.transpose(p['w2'])

    full = lambda r, c: pl.BlockSpec((r, c), lambda i: (0, 0))
    tile = lambda c: pl.BlockSpec((_BN, c), lambda i: (i, 0))
    tile3 = lambda c: pl.BlockSpec((K, _BN, c), lambda i: (0, i, 0))
    stat = lambda c: pl.BlockSpec((8, c), lambda i: (0, 0))
    sds = lambda *s: jax.ShapeDtypeStruct(s, jnp.float32)

    y1, st1, sts = _pcall(
        _k1_body, grid=grid,
        in_specs=[tile(8), full(8, 8), full(8, 32)],
        out_specs=[tile(8), stat(8), stat(32)],
        out_shape=[sds(N, 8), sds(8, 8), sds(8, 32)],
    )(feat, w1t, wst)
    a1, c1 = _bn_affine(st1, p['g1'], p['be1'], N)
    as_, cs_ = _bn_affine(sts, p['gs'], p['bes'], N)

    t1 = _pcall(
        _k2_body, grid=grid,
        in_specs=[tile(8), tile(3), full(1, 8), full(1, 8)],
        out_specs=tile(16),
        out_shape=sds(N, 16),
    )(y1, xyz0, a1, c1)

    g1 = _sc_gather(t1, idxt).reshape(K, N, 16)

    stb1 = _pcall(
        functools.partial(_k3_body, K=K), grid=grid,
        in_specs=[tile3(16), tile(16), full(16, 8)],
        out_specs=stat(8),
        out_shape=sds(8, 8),
    )(g1, t1, wb1t)
    ab1, cb1 = _bn_affine(stb1, p['bb_g1'], p['bb_be1'], N * K)

    yc, yd, stc, std_ = _pcall(
        functools.partial(_k4_body, K=K), grid=grid,
        in_specs=[tile3(16), tile(16), full(16, 8), full(1, 8), full(1, 8),
                  full(16, 16), full(16, 8), full(8, 8)],
        out_specs=[tile(8), tile3(8), stat(8), stat(8)],
        out_shape=[sds(N, 8), sds(K, N, 8), sds(8, 8), sds(8, 8)],
    )(g1, t1, wb1t, ab1, cb1, fct1, apw1t, wdt)
    ac, cc = _bn_affine(stc, p['ap1_g'], p['ap1_be'], N)
    ad, cd = _bn_affine(std_, p['bb_g2'], p['bb_be2'], N * K)

    t2 = _pcall(
        _k5_body, grid=grid,
        in_specs=[tile(8), full(1, 8), full(1, 8)],
        out_specs=tile(8),
        out_shape=sds(N, 8),
    )(yc, ac, cc)

    g2 = _sc_gather(t2, idxt).reshape(K, N, 8)

    ye, ste = _pcall(
        functools.partial(_k6_body, K=K), grid=grid,
        in_specs=[tile3(8), tile3(8), full(1, 8), full(1, 8),
                  full(16, 16), full(16, 16)],
        out_specs=[tile(16), stat(16)],
        out_shape=[sds(N, 16), sds(8, 16)],
    )(g2, yd, ad, cd, fct2, apw2t)
    ae, ce = _bn_affine(ste, p['ap2_g'], p['ap2_be'], N)

    yf, stf = _pcall(
        _k7_body, grid=grid,
        in_specs=[tile(16), full(1, 16), full(1, 16), full(16, 32)],
        out_specs=[tile(32), stat(32)],
        out_shape=[sds(N, 32), sds(8, 32)],
    )(ye, ae, ce, w2t)
    af, cf = _bn_affine(stf, p['g2'], p['be2'], N)

    out_nc = _pcall(
        _k8_body, grid=grid,
        in_specs=[tile(32), tile(8), full(8, 32), full(1, 32), full(1, 32),
                  full(1, 32), full(1, 32)],
        out_specs=tile(32),
        out_shape=sds(N, 32),
    )(yf, feat, wst, af, cf, as_, cs_)

    return jnp.transpose(out_nc)[None, :, :, None]


# speedup vs baseline: 22.2278x; 2.8728x over previous
import functools

import jax
import jax.numpy as jnp
from jax import lax
from jax.experimental import pallas as pl
from jax.experimental.pallas import tpu as pltpu
from jax.experimental.pallas import tpu_sc as plsc

_pcall = pl.pallas_call


def _leaky(x, slope):
    return jnp.where(x >= 0, x, slope * x)


def _sc_gather(table, idx):
    M = idx.shape[0]
    D = table.shape[1]
    info = plsc.get_sparse_core_info()
    nw = info.num_cores * info.num_subcores
    cs = 5000
    n_chunks = M // cs
    mesh = plsc.VectorSubcoreMesh(core_axis_name="c", subcore_axis_name="s")

    @functools.partial(
        pl.kernel,
        mesh=mesh,
        compiler_params=pltpu.CompilerParams(use_tc_tiling_on_sc=False),
        out_type=jax.ShapeDtypeStruct((M, D), jnp.float32),
        scratch_types=[
            pltpu.VMEM((cs,), jnp.int32),
            pltpu.VMEM((cs, D), jnp.float32),
            pltpu.SemaphoreType.DMA,
        ],
    )
    def k(table_hbm, idx_hbm, out_hbm, idx_v, rows_v, sem):
        wid = lax.axis_index("s") * info.num_cores + lax.axis_index("c")

        def body(j, carry):
            cid = j * nw + wid

            @pl.when(cid < n_chunks)
            def _():
                off = cid * cs
                pltpu.sync_copy(idx_hbm.at[pl.ds(off, cs)], idx_v)
                pltpu.async_copy(table_hbm.at[idx_v], rows_v, sem).wait()
                pltpu.sync_copy(rows_v, out_hbm.at[pl.ds(off, cs)])

            return carry

        lax.fori_loop(0, pl.cdiv(n_chunks, nw), body, 0)

    return k(table, idx)


def _acc(ref, upd):
    @pl.when(pl.program_id(0) == 0)
    def _():
        ref[...] = upd

    @pl.when(pl.program_id(0) > 0)
    def _():
        ref[...] += upd


def _stat_rows(y, c):
    return jnp.concatenate(
        [jnp.sum(y, 1)[None], jnp.sum(y * y, 1)[None],
         jnp.zeros((6, c), jnp.float32)], 0)


def _bn_affine(st, g, be, count):
    m = st[0] / count
    v = st[1] / count - m * m
    a = g / jnp.sqrt(v + 1e-6)
    return (a[:, None], (be - a * m)[:, None])


def _k1_body(feat_ref, w1_ref, ws_ref, y1_ref, st1_ref, sts_ref):
    x = feat_ref[...]
    y1 = jnp.dot(w1_ref[...], x, preferred_element_type=jnp.float32)
    ys = jnp.dot(ws_ref[...], x, preferred_element_type=jnp.float32)
    y1_ref[...] = y1
    st1_ref[...] = _stat_rows(y1, 8)
    sts_ref[...] = _stat_rows(ys, 32)


def _k2_body(y1_ref, xyz_ref, a_ref, c_ref, t1_ref):
    fpc = _leaky(a_ref[...] * y1_ref[...] + c_ref[...], 0.2)
    t1_ref[...] = jnp.concatenate([fpc, xyz_ref[...]], axis=0)


def _fxyz_k(gk, rep, n):
    nx = gk[8:11, :]
    rel = rep - nx
    dist = jnp.sqrt(jnp.sum(rel * rel, axis=0, keepdims=True))
    return jnp.concatenate(
        [dist, rel, rep, nx, jnp.zeros((6, n), jnp.float32)], axis=0)


def _k3_body(g1_ref, xyz_ref, wb1_ref, st_ref):
    n = g1_ref.shape[2]
    fx = _fxyz_k(g1_ref[0], xyz_ref[0:3, :], n)
    yb1 = jnp.dot(wb1_ref[...], fx, preferred_element_type=jnp.float32)
    _acc(st_ref, _stat_rows(yb1, 8))


def _k4_body(g1_ref, xyz_ref, wb1_ref, ab_ref, cb_ref, fct_ref, apw_ref,
             wd_ref, yc_ref, yd_ref, stc_ref, std_ref, m_s, z_s, s_s):
    k = pl.program_id(0)
    nk = pl.num_programs(0)
    n = g1_ref.shape[2]
    gk = g1_ref[0]
    fx = _fxyz_k(gk, xyz_ref[0:3, :], n)
    yb1 = jnp.dot(wb1_ref[...], fx, preferred_element_type=jnp.float32)
    fxyz1 = _leaky(ab_ref[...] * yb1 + cb_ref[...], 0.2)
    fck = jnp.concatenate([gk[0:8, :], fxyz1], axis=0)
    att = jnp.dot(fct_ref[...], fck, preferred_element_type=jnp.float32)
    ydk = jnp.dot(wd_ref[...], fxyz1, preferred_element_type=jnp.float32)
    yd_ref[0] = ydk
    _acc(std_ref, _stat_rows(ydk, 8))

    @pl.when(k == 0)
    def _():
        m_s[...] = att
        z_s[...] = jnp.ones_like(att)
        s_s[...] = fck

    @pl.when(k > 0)
    def _():
        m_new = jnp.maximum(m_s[...], att)
        r = jnp.exp(m_s[...] - m_new)
        e = jnp.exp(att - m_new)
        z_s[...] = z_s[...] * r + e
        s_s[...] = s_s[...] * r + fck * e
        m_s[...] = m_new

    @pl.when(k == nk - 1)
    def _():
        s1 = s_s[...] / z_s[...]
        yc = jnp.dot(apw_ref[...], s1, preferred_element_type=jnp.float32)
        yc_ref[...] = yc
        stc_ref[...] = _stat_rows(yc, 8)


def _k5_body(yc_ref, a_ref, c_ref, t2_ref):
    t2_ref[...] = _leaky(a_ref[...] * yc_ref[...] + c_ref[...], 0.2)


def _k6_body(g2_ref, yd_ref, ad_ref, cd_ref, fct_ref, apw_ref,
             ye_ref, ste_ref, m_s, z_s, s_s):
    k = pl.program_id(0)
    nk = pl.num_programs(0)
    fxyz2 = _leaky(ad_ref[...] * yd_ref[0] + cd_ref[...], 0.2)
    fck = jnp.concatenate([g2_ref[0], fxyz2], axis=0)
    att = jnp.dot(fct_ref[...], fck, preferred_element_type=jnp.float32)

    @pl.when(k == 0)
    def _():
        m_s[...] = att
        z_s[...] = jnp.ones_like(att)
        s_s[...] = fck

    @pl.when(k > 0)
    def _():
        m_new = jnp.maximum(m_s[...], att)
        r = jnp.exp(m_s[...] - m_new)
        e = jnp.exp(att - m_new)
        z_s[...] = z_s[...] * r + e
        s_s[...] = s_s[...] * r + fck * e
        m_s[...] = m_new

    @pl.when(k == nk - 1)
    def _():
        s2 = s_s[...] / z_s[...]
        ye = jnp.dot(apw_ref[...], s2, preferred_element_type=jnp.float32)
        ye_ref[...] = ye
        ste_ref[...] = _stat_rows(ye, 16)


def _k7_body(ye_ref, ae_ref, ce_ref, w2_ref, yf_ref, stf_ref):
    fpc2 = _leaky(ae_ref[...] * ye_ref[...] + ce_ref[...], 0.2)
    yf = jnp.dot(w2_ref[...], fpc2, preferred_element_type=jnp.float32)
    yf_ref[...] = yf
    stf_ref[...] = _stat_rows(yf, 32)


def _k8_body(yf_ref, feat_ref, ws_ref, af_ref, cf_ref, as_ref, cs_ref,
             out_ref):
    ys = jnp.dot(ws_ref[...], feat_ref[...], preferred_element_type=jnp.float32)
    pre = (af_ref[...] * yf_ref[...] + cf_ref[...]
           + as_ref[...] * ys + cs_ref[...])
    out_ref[...] = _leaky(pre, 0.01)


def kernel(feature, xyz, neigh_idx, p):
    N = feature.shape[2]
    K = neigh_idx.shape[2]

    feat = feature[0, :, :, 0]
    xyz8 = jnp.concatenate(
        [jnp.transpose(xyz[0]), jnp.zeros((5, N), jnp.float32)], 0)
    idxt = jnp.transpose(neigh_idx[0]).astype(jnp.int32).reshape(-1)

    wb1 = jnp.concatenate([p['bb_w1'], jnp.zeros((8, 6), jnp.float32)], 1)

    plane = lambda c: pl.BlockSpec((1, c, N), lambda k: (k, 0, 0))
    const2 = lambda r, c: pl.BlockSpec((r, c), lambda k: (0, 0))
    sds = lambda *s: jax.ShapeDtypeStruct(s, jnp.float32)
    vs = lambda *s: pltpu.VMEM(s, jnp.float32)

    y1, st1, sts = _pcall(
        _k1_body,
        out_shape=[sds(8, N), sds(8, 8), sds(8, 32)],
    )(feat, p['w1'], p['ws'])
    a1, c1 = _bn_affine(st1, p['g1'], p['be1'], N)
    as_, cs_ = _bn_affine(sts, p['gs'], p['bes'], N)

    t1 = _pcall(
        _k2_body,
        out_shape=sds(16, N),
    )(y1, xyz8, a1, c1)

    g1 = _sc_gather(jnp.transpose(t1), idxt)
    g1t = jnp.transpose(g1.reshape(K, N, 16), (0, 2, 1))

    stb1 = _pcall(
        _k3_body, grid=(K,),
        in_specs=[plane(16), const2(8, N), const2(8, 16)],
        out_specs=const2(8, 8),
        out_shape=sds(8, 8),
    )(g1t, xyz8, wb1)
    ab1, cb1 = _bn_affine(stb1, p['bb_g1'], p['bb_be1'], N * K)

    yc, yd, stc, std_ = _pcall(
        _k4_body, grid=(K,),
        in_specs=[plane(16), const2(8, N), const2(8, 16), const2(8, 1),
                  const2(8, 1), const2(16, 16), const2(8, 16), const2(8, 8)],
        out_specs=[const2(8, N), plane(8), const2(8, 8), const2(8, 8)],
        out_shape=[sds(8, N), sds(K, 8, N), sds(8, 8), sds(8, 8)],
        scratch_shapes=[vs(16, N), vs(16, N), vs(16, N)],
    )(g1t, xyz8, wb1, ab1, cb1, p['ap1_fc'], p['ap1_w'], p['bb_w2'])
    ac, cc = _bn_affine(stc, p['ap1_g'], p['ap1_be'], N)
    ad, cd = _bn_affine(std_, p['bb_g2'], p['bb_be2'], N * K)

    t2 = _pcall(
        _k5_body,
        out_shape=sds(8, N),
    )(yc, ac, cc)

    g2 = _sc_gather(jnp.transpose(t2), idxt)
    g2t = jnp.transpose(g2.reshape(K, N, 8), (0, 2, 1))

    ye, ste = _pcall(
        _k6_body, grid=(K,),
        in_specs=[plane(8), plane(8), const2(8, 1), const2(8, 1),
                  const2(16, 16), const2(16, 16)],
        out_specs=[const2(16, N), const2(8, 16)],
        out_shape=[sds(16, N), sds(8, 16)],
        scratch_shapes=[vs(16, N), vs(16, N), vs(16, N)],
    )(g2t, yd, ad, cd, p['ap2_fc'], p['ap2_w'])
    ae, ce = _bn_affine(ste, p['ap2_g'], p['ap2_be'], N)

    yf, stf = _pcall(
        _k7_body,
        out_shape=[sds(32, N), sds(8, 32)],
    )(ye, ae, ce, p['w2'])
    af, cf = _bn_affine(stf, p['g2'], p['be2'], N)

    out_cn = _pcall(
        _k8_body,
        out_shape=sds(32, N),
    )(yf, feat, p['ws'], af, cf, as_, cs_)

    return out_cn[None, :, :, None]
